# baseline (speedup 1.0000x reference)
import jax
import jax.numpy as jnp
from jax.experimental import pallas as pl
from jax.experimental.pallas import tpu as pltpu

_ROWS_PER_BLK = 512


def _fused_body(mu_ref, ls_ref, eps_ref, mu2_ref, ls2_ref, eps2_ref,
                x_ref, y_ref, probs_ref, mask_scr):
    i = pl.program_id(0)
    d = mu_ref.shape[1]
    k = int(d * 0.7)

    @pl.when(i == 0)
    def _():
        std = jnp.exp(0.5 * ls_ref[...])
        z = mu_ref[...] + eps_ref[...] * std
        p = 1.0 / (1.0 + jnp.exp(-z))
        probs_ref[...] = p
        bits = jax.lax.bitcast_convert_type(p, jnp.int32)

        std2 = jnp.exp(0.5 * ls2_ref[...])
        z2 = mu2_ref[...] + eps2_ref[...] * std2
        p2 = 1.0 / (1.0 + jnp.exp(-z2))
        bits2 = jax.lax.bitcast_convert_type(p2, jnp.int32)

        def step(t, prefix):
            cand = prefix | (1 << (30 - t))
            cnt = jnp.sum((bits2 < cand).astype(jnp.int32))
            return jnp.where(cnt <= k, cand, prefix)

        thr_bits = jax.lax.fori_loop(0, 31, step, jnp.int32(0))
        mask_scr[...] = (bits > thr_bits).astype(jnp.float32)

    @pl.when(i > 0)
    def _():
        y_ref[...] = x_ref[...] * mask_scr[...]


def kernel(x, mu, log_sigma, eps):
    b, s, d = x.shape
    rows = b * s
    x2 = x.reshape(rows, d)
    mu1 = mu.reshape(1, d)
    ls1 = log_sigma.reshape(1, d)
    eps1 = eps.reshape(1, d)
    r = d // 128
    mu2 = mu.reshape(r, 128)
    ls2 = log_sigma.reshape(r, 128)
    eps2 = eps.reshape(r, 128)

    grid = (rows // _ROWS_PER_BLK + 1,)
    prev = lambda i: (jnp.maximum(i - 1, 0), 0)
    y, probs = pl.pallas_call(
        _fused_body,
        grid=grid,
        in_specs=[
            pl.BlockSpec((1, d), lambda i: (0, 0)),
            pl.BlockSpec((1, d), lambda i: (0, 0)),
            pl.BlockSpec((1, d), lambda i: (0, 0)),
            pl.BlockSpec((r, 128), lambda i: (0, 0)),
            pl.BlockSpec((r, 128), lambda i: (0, 0)),
            pl.BlockSpec((r, 128), lambda i: (0, 0)),
            pl.BlockSpec((_ROWS_PER_BLK, d), prev),
        ],
        out_specs=[
            pl.BlockSpec((_ROWS_PER_BLK, d), prev),
            pl.BlockSpec((1, d), lambda i: (0, 0)),
        ],
        out_shape=[
            jax.ShapeDtypeStruct((rows, d), jnp.float32),
            jax.ShapeDtypeStruct((1, d), jnp.float32),
        ],
        scratch_shapes=[pltpu.VMEM((1, d), jnp.float32)],
    )(mu1, ls1, eps1, mu2, ls2, eps2, x2)
    return y.reshape(b, s, d), probs.reshape(d)


# speedup vs baseline: 1.0114x; 1.0114x over previous
import jax
import jax.numpy as jnp
from jax.experimental import pallas as pl
from jax.experimental.pallas import tpu as pltpu

_ROWS_PER_BLK = 512


def _fused_body(mu_ref, ls_ref, eps_ref, mu2_ref, ls2_ref, eps2_ref,
                x_ref, y_ref, probs_ref, mask_scr):
    i = pl.program_id(0)
    d = mu_ref.shape[1]
    k = int(d * 0.7)

    @pl.when(i == 0)
    def _():
        std = jnp.exp(0.5 * ls_ref[...])
        z = mu_ref[...] + eps_ref[...] * std
        p = 1.0 / (1.0 + jnp.exp(-z))
        probs_ref[...] = p
        bits = jax.lax.bitcast_convert_type(p, jnp.int32)

        std2 = jnp.exp(0.5 * ls2_ref[...])
        z2 = mu2_ref[...] + eps2_ref[...] * std2
        p2 = 1.0 / (1.0 + jnp.exp(-z2))
        bits2 = jax.lax.bitcast_convert_type(p2, jnp.int32)

        prefix = jnp.int32(0)
        for t in range(31):
            cand = prefix | jnp.int32(1 << (30 - t))
            cnt = jnp.sum((bits2 < cand).astype(jnp.int32))
            prefix = jnp.where(cnt <= k, cand, prefix)
        thr_bits = prefix
        mask_scr[...] = (bits > thr_bits).astype(jnp.float32)

    y_ref[...] = x_ref[...] * mask_scr[...]


def kernel(x, mu, log_sigma, eps):
    b, s, d = x.shape
    rows = b * s
    x2 = x.reshape(rows, d)
    mu1 = mu.reshape(1, d)
    ls1 = log_sigma.reshape(1, d)
    eps1 = eps.reshape(1, d)
    r = d // 128
    mu2 = mu.reshape(r, 128)
    ls2 = log_sigma.reshape(r, 128)
    eps2 = eps.reshape(r, 128)

    grid = (rows // _ROWS_PER_BLK,)
    y, probs = pl.pallas_call(
        _fused_body,
        grid=grid,
        in_specs=[
            pl.BlockSpec((1, d), lambda i: (0, 0)),
            pl.BlockSpec((1, d), lambda i: (0, 0)),
            pl.BlockSpec((1, d), lambda i: (0, 0)),
            pl.BlockSpec((r, 128), lambda i: (0, 0)),
            pl.BlockSpec((r, 128), lambda i: (0, 0)),
            pl.BlockSpec((r, 128), lambda i: (0, 0)),
            pl.BlockSpec((_ROWS_PER_BLK, d), lambda i: (i, 0)),
        ],
        out_specs=[
            pl.BlockSpec((_ROWS_PER_BLK, d), lambda i: (i, 0)),
            pl.BlockSpec((1, d), lambda i: (0, 0)),
        ],
        out_shape=[
            jax.ShapeDtypeStruct((rows, d), jnp.float32),
            jax.ShapeDtypeStruct((1, d), jnp.float32),
        ],
        scratch_shapes=[pltpu.VMEM((1, d), jnp.float32)],
    )(mu1, ls1, eps1, mu2, ls2, eps2, x2)
    return y.reshape(b, s, d), probs.reshape(d)


# speedup vs baseline: 1.0223x; 1.0107x over previous
import jax
import jax.numpy as jnp
from jax.experimental import pallas as pl
from jax.experimental.pallas import tpu as pltpu

_ROWS_PER_BLK = 512


def _fused_body(mu_ref, ls_ref, eps_ref, mu2_ref, ls2_ref, eps2_ref,
                x_ref, y_ref, probs_ref, mask_scr):
    i = pl.program_id(0)
    d = mu_ref.shape[1]
    k = int(d * 0.7)

    @pl.when(i == 0)
    def _():
        std = jnp.exp(0.5 * ls_ref[...])
        z = mu_ref[...] + eps_ref[...] * std
        p = 1.0 / (1.0 + jnp.exp(-z))
        probs_ref[...] = p
        bits = jax.lax.bitcast_convert_type(p, jnp.int32)

        std2 = jnp.exp(0.5 * ls2_ref[...])
        z2 = mu2_ref[...] + eps2_ref[...] * std2
        p2 = 1.0 / (1.0 + jnp.exp(-z2))
        bits2 = jax.lax.bitcast_convert_type(p2, jnp.int32)

        def count(c):
            return jnp.sum((bits2 < c).astype(jnp.int32))

        prefix = jnp.int32(0)
        for t in range(15):
            hi = jnp.int32(1 << (30 - 2 * t))
            lo = jnp.int32(1 << (29 - 2 * t))
            a = prefix | hi
            bq = prefix | lo
            c = a | lo
            cnt_a, cnt_b, cnt_c = count(a), count(bq), count(c)
            keep_a = cnt_a <= k
            prefix = jnp.where(keep_a, a, prefix)
            cand2 = jnp.where(keep_a, c, bq)
            cnt2 = jnp.where(keep_a, cnt_c, cnt_b)
            prefix = jnp.where(cnt2 <= k, cand2, prefix)
        last = prefix | jnp.int32(1)
        thr_bits = jnp.where(count(last) <= k, last, prefix)
        mask_scr[...] = (bits > thr_bits).astype(jnp.float32)

    y_ref[...] = x_ref[...] * mask_scr[...]


def kernel(x, mu, log_sigma, eps):
    b, s, d = x.shape
    rows = b * s
    x2 = x.reshape(rows, d)
    mu1 = mu.reshape(1, d)
    ls1 = log_sigma.reshape(1, d)
    eps1 = eps.reshape(1, d)
    r = d // 128
    mu2 = mu.reshape(r, 128)
    ls2 = log_sigma.reshape(r, 128)
    eps2 = eps.reshape(r, 128)

    grid = (rows // _ROWS_PER_BLK,)
    y, probs = pl.pallas_call(
        _fused_body,
        grid=grid,
        in_specs=[
            pl.BlockSpec((1, d), lambda i: (0, 0)),
            pl.BlockSpec((1, d), lambda i: (0, 0)),
            pl.BlockSpec((1, d), lambda i: (0, 0)),
            pl.BlockSpec((r, 128), lambda i: (0, 0)),
            pl.BlockSpec((r, 128), lambda i: (0, 0)),
            pl.BlockSpec((r, 128), lambda i: (0, 0)),
            pl.BlockSpec((_ROWS_PER_BLK, d), lambda i: (i, 0)),
        ],
        out_specs=[
            pl.BlockSpec((_ROWS_PER_BLK, d), lambda i: (i, 0)),
            pl.BlockSpec((1, d), lambda i: (0, 0)),
        ],
        out_shape=[
            jax.ShapeDtypeStruct((rows, d), jnp.float32),
            jax.ShapeDtypeStruct((1, d), jnp.float32),
        ],
        scratch_shapes=[pltpu.VMEM((1, d), jnp.float32)],
    )(mu1, ls1, eps1, mu2, ls2, eps2, x2)
    return y.reshape(b, s, d), probs.reshape(d)


# speedup vs baseline: 1.0252x; 1.0028x over previous
import jax
import jax.numpy as jnp
from jax.experimental import pallas as pl
from jax.experimental.pallas import tpu as pltpu

_ROWS_PER_BLK = 512


def _fused_body(mu_ref, ls_ref, eps_ref, mu2_ref, ls2_ref, eps2_ref,
                x_ref, y_ref, probs_ref, mask_scr):
    i = pl.program_id(0)
    d = mu_ref.shape[1]
    k = int(d * 0.7)

    @pl.when(i == 0)
    def _():
        std = jnp.exp(0.5 * ls_ref[...])
        z = mu_ref[...] + eps_ref[...] * std
        p = 1.0 / (1.0 + jnp.exp(-z))
        probs_ref[...] = p
        bits = jax.lax.bitcast_convert_type(p, jnp.int32)

        std2 = jnp.exp(0.5 * ls2_ref[...])
        z2 = mu2_ref[...] + eps2_ref[...] * std2
        p2 = 1.0 / (1.0 + jnp.exp(-z2))
        bits2 = jax.lax.bitcast_convert_type(p2, jnp.int32)

        def count(c):
            return jnp.sum((bits2 < c).astype(jnp.int32))

        prefix = jnp.int32(0)
        for t in range(10):
            m2 = jnp.int32(1 << (30 - 3 * t))
            m1 = jnp.int32(1 << (29 - 3 * t))
            m0 = jnp.int32(1 << (28 - 3 * t))
            c2 = count(prefix | m2)
            c1a = count(prefix | m1)
            c1b = count(prefix | m2 | m1)
            c0aa = count(prefix | m0)
            c0ab = count(prefix | m1 | m0)
            c0ba = count(prefix | m2 | m0)
            c0bb = count(prefix | m2 | m1 | m0)
            keep2 = c2 <= k
            prefix = jnp.where(keep2, prefix | m2, prefix)
            cnt1 = jnp.where(keep2, c1b, c1a)
            keep1 = cnt1 <= k
            prefix = jnp.where(keep1, prefix | m1, prefix)
            cnt0 = jnp.where(keep2,
                             jnp.where(keep1, c0bb, c0ba),
                             jnp.where(keep1, c0ab, c0aa))
            prefix = jnp.where(cnt0 <= k, prefix | m0, prefix)
        last = prefix | jnp.int32(1)
        thr_bits = jnp.where(count(last) <= k, last, prefix)
        mask_scr[...] = (bits > thr_bits).astype(jnp.float32)

    y_ref[...] = x_ref[...] * mask_scr[...]


def kernel(x, mu, log_sigma, eps):
    b, s, d = x.shape
    rows = b * s
    x2 = x.reshape(rows, d)
    mu1 = mu.reshape(1, d)
    ls1 = log_sigma.reshape(1, d)
    eps1 = eps.reshape(1, d)
    r = d // 128
    mu2 = mu.reshape(r, 128)
    ls2 = log_sigma.reshape(r, 128)
    eps2 = eps.reshape(r, 128)

    grid = (rows // _ROWS_PER_BLK,)
    y, probs = pl.pallas_call(
        _fused_body,
        grid=grid,
        in_specs=[
            pl.BlockSpec((1, d), lambda i: (0, 0)),
            pl.BlockSpec((1, d), lambda i: (0, 0)),
            pl.BlockSpec((1, d), lambda i: (0, 0)),
            pl.BlockSpec((r, 128), lambda i: (0, 0)),
            pl.BlockSpec((r, 128), lambda i: (0, 0)),
            pl.BlockSpec((r, 128), lambda i: (0, 0)),
            pl.BlockSpec((_ROWS_PER_BLK, d), lambda i: (i, 0)),
        ],
        out_specs=[
            pl.BlockSpec((_ROWS_PER_BLK, d), lambda i: (i, 0)),
            pl.BlockSpec((1, d), lambda i: (0, 0)),
        ],
        out_shape=[
            jax.ShapeDtypeStruct((rows, d), jnp.float32),
            jax.ShapeDtypeStruct((1, d), jnp.float32),
        ],
        scratch_shapes=[pltpu.VMEM((1, d), jnp.float32)],
    )(mu1, ls1, eps1, mu2, ls2, eps2, x2)
    return y.reshape(b, s, d), probs.reshape(d)


# speedup vs baseline: 1.0265x; 1.0013x over previous
import jax
import jax.numpy as jnp
from jax.experimental import pallas as pl
from jax.experimental.pallas import tpu as pltpu

_ROWS_PER_BLK = 512


def _fused_body(mu_ref, ls_ref, eps_ref, mu2_ref, ls2_ref, eps2_ref,
                x_ref, y_ref, probs_ref, mask_scr):
    i = pl.program_id(0)
    d = mu_ref.shape[1]
    k = int(d * 0.7)

    @pl.when(i == 0)
    def _():
        std = jnp.exp(0.5 * ls_ref[...])
        z = mu_ref[...] + eps_ref[...] * std
        p = 1.0 / (1.0 + jnp.exp(-z))
        probs_ref[...] = p
        bits = jax.lax.bitcast_convert_type(p, jnp.int32)

        std2 = jnp.exp(0.5 * ls2_ref[...])
        z2 = mu2_ref[...] + eps2_ref[...] * std2
        p2 = 1.0 / (1.0 + jnp.exp(-z2))
        bits2 = jax.lax.bitcast_convert_type(p2, jnp.int32)

        def count(c):
            return jnp.sum((bits2 < c).astype(jnp.int32))

        prefix = jnp.int32(0)
        groups = [(30, 29, 28, 27), (26, 25, 24, 23), (22, 21, 20, 19),
                  (18, 17, 16, 15), (14, 13, 12, 11), (10, 9, 8, 7),
                  (6, 5, 4, 3), (2, 1, 0)]
        for positions in groups:
            nb = len(positions)
            masks = [jnp.int32(1 << p) for p in positions]
            cnts = {}
            for sel in range(1, 1 << nb):
                cand = prefix
                for j in range(nb):
                    if (sel >> (nb - 1 - j)) & 1:
                        cand = cand | masks[j]
                cnts[sel] = count(cand)
            path = jnp.int32(0)
            for j in range(nb):
                cnt_j = jnp.int32(0)
                for concrete in range(1 << j):
                    sel = (concrete * 2 + 1) << (nb - 1 - j)
                    cnt_j = jnp.where(path == concrete, cnts[sel], cnt_j)
                take = cnt_j <= k
                prefix = jnp.where(take, prefix | masks[j], prefix)
                path = jnp.where(take, path * 2 + 1, path * 2)
        thr_bits = prefix
        mask_scr[...] = (bits > thr_bits).astype(jnp.float32)

    y_ref[...] = x_ref[...] * mask_scr[...]


def kernel(x, mu, log_sigma, eps):
    b, s, d = x.shape
    rows = b * s
    x2 = x.reshape(rows, d)
    mu1 = mu.reshape(1, d)
    ls1 = log_sigma.reshape(1, d)
    eps1 = eps.reshape(1, d)
    r = d // 128
    mu2 = mu.reshape(r, 128)
    ls2 = log_sigma.reshape(r, 128)
    eps2 = eps.reshape(r, 128)

    grid = (rows // _ROWS_PER_BLK,)
    y, probs = pl.pallas_call(
        _fused_body,
        grid=grid,
        in_specs=[
            pl.BlockSpec((1, d), lambda i: (0, 0)),
            pl.BlockSpec((1, d), lambda i: (0, 0)),
            pl.BlockSpec((1, d), lambda i: (0, 0)),
            pl.BlockSpec((r, 128), lambda i: (0, 0)),
            pl.BlockSpec((r, 128), lambda i: (0, 0)),
            pl.BlockSpec((r, 128), lambda i: (0, 0)),
            pl.BlockSpec((_ROWS_PER_BLK, d), lambda i: (i, 0)),
        ],
        out_specs=[
            pl.BlockSpec((_ROWS_PER_BLK, d), lambda i: (i, 0)),
            pl.BlockSpec((1, d), lambda i: (0, 0)),
        ],
        out_shape=[
            jax.ShapeDtypeStruct((rows, d), jnp.float32),
            jax.ShapeDtypeStruct((1, d), jnp.float32),
        ],
        scratch_shapes=[pltpu.VMEM((1, d), jnp.float32)],
    )(mu1, ls1, eps1, mu2, ls2, eps2, x2)
    return y.reshape(b, s, d), probs.reshape(d)
